# baseline (speedup 1.0000x reference)
import functools

import jax
import jax.numpy as jnp
from jax.experimental import pallas as pl
from jax.experimental.pallas import tpu as pltpu

_B, _C, _H, _W = 8, 96, 56, 56
_HW = _H * _W
_HID = _C // 16
_BC = _B * _C
_N = 8
_AHEAD = _N // 2


def _perm_const():
    pkey = jax.random.key(42)
    return jax.random.permutation(jax.random.fold_in(pkey, 0), _C)


def _index_body(x_ref, w1_ref, b1_ref, w2_ref, b2_ref, perm_ref, ind_ref):
    s = jnp.mean(x_ref[...], axis=(2, 3))
    h = jnp.maximum(
        jax.lax.dot_general(s, w1_ref[...], (((1,), (1,)), ((), ())))
        + b1_ref[...], 0.0)
    lg = jax.lax.dot_general(h, w2_ref[...], (((1,), (1,)), ((), ())))
    sc = jax.nn.sigmoid(lg + b2_ref[...])

    gt = (sc[:, None, :] > sc[:, :, None])
    eq = (sc[:, None, :] == sc[:, :, None])
    ii = jax.lax.broadcasted_iota(jnp.int32, (_B, _C, _C), 1)
    jj = jax.lax.broadcasted_iota(jnp.int32, (_B, _C, _C), 2)
    r = jnp.sum((gt | (eq & (jj < ii))).astype(jnp.int32), axis=2)

    match = (r[:, :, None] == perm_ref[...][0][None, None, :])
    ci = jax.lax.broadcasted_iota(jnp.int32, (_B, _C, _C), 1)
    sg = jnp.sum(jnp.where(match, ci, 0), axis=1)

    base = jax.lax.broadcasted_iota(jnp.int32, (_B, _B, _C), 0) * _C
    ind_ref[...] = base + sg[None, :, :]


def _gather_ring_body(sg_ref, x_ref, o_ref, buf, in_sems, out_sems):
    def in_copy(k, slot):
        return pltpu.make_async_copy(
            x_ref.at[:, pl.ds(sg_ref[k], 1)], buf.at[slot], in_sems.at[slot])

    def out_copy(k, slot):
        return pltpu.make_async_copy(
            buf.at[slot],
            o_ref.at[:, pl.ds(k // _C, 1), pl.ds(jax.lax.rem(k, _C), 1)],
            out_sems.at[slot])

    for t in range(_AHEAD):
        in_copy(t, t).start()

    def step(k, _):
        slot = jax.lax.rem(k, _N)
        in_copy(k, slot).wait()
        out_copy(k, slot).start()

        t = k + _AHEAD
        tslot = jax.lax.rem(t, _N)

        @pl.when(t < _BC)
        def _issue_next():
            @pl.when(t >= _N)
            def _free_buf():
                out_copy(t - _N, tslot).wait()

            in_copy(t, tslot).start()

        return 0

    jax.lax.fori_loop(0, _BC, step, 0, unroll=2)

    def drain(k, _):
        out_copy(k, jax.lax.rem(k, _N)).wait()
        return 0

    jax.lax.fori_loop(_BC - _N, _BC, drain, 0)


@jax.jit
def kernel(x, W1, b1, W2, b2):
    perm = _perm_const().astype(jnp.int32).reshape(1, _C)

    ind = pl.pallas_call(
        _index_body,
        out_shape=jax.ShapeDtypeStruct((_B, _B, _C), jnp.int32),
    )(x, W1, b1.reshape(1, _HID), W2, b2.reshape(1, _C), perm)

    sg_flat = ind[0].reshape(_BC)

    out = pl.pallas_call(
        _gather_ring_body,
        in_specs=[
            pl.BlockSpec(memory_space=pltpu.MemorySpace.SMEM),
            pl.BlockSpec(memory_space=pltpu.MemorySpace.HBM),
        ],
        out_specs=pl.BlockSpec(memory_space=pltpu.MemorySpace.HBM),
        out_shape=jax.ShapeDtypeStruct((_B, _B, _C, _H, _W), jnp.float32),
        scratch_shapes=[
            pltpu.VMEM((_N, _B, 1, 1, _H, _W), jnp.float32),
            pltpu.SemaphoreType.DMA((_N,)),
            pltpu.SemaphoreType.DMA((_N,)),
        ],
    )(sg_flat, x[:, :, None])

    return out


# speedup vs baseline: 1.6098x; 1.6098x over previous
import functools

import jax
import jax.numpy as jnp
from jax.experimental import pallas as pl
from jax.experimental.pallas import tpu as pltpu

_B, _C, _H, _W = 8, 96, 56, 56
_HW = _H * _W
_HID = _C // 16
_BC = _B * _C
_N = 8
_AHEAD = _N // 2


def _perm_const():
    pkey = jax.random.key(42)
    return jax.random.permutation(jax.random.fold_in(pkey, 0), _C)


def _index_body(x_ref, w1_ref, b1_ref, w2_ref, b2_ref, perm_ref, ind_ref):
    s = jnp.mean(x_ref[...], axis=(2, 3))
    h = jnp.maximum(
        jax.lax.dot_general(s, w1_ref[...], (((1,), (1,)), ((), ())))
        + b1_ref[...], 0.0)
    lg = jax.lax.dot_general(h, w2_ref[...], (((1,), (1,)), ((), ())))
    sc = jax.nn.sigmoid(lg + b2_ref[...])

    gt = (sc[:, None, :] > sc[:, :, None])
    eq = (sc[:, None, :] == sc[:, :, None])
    ii = jax.lax.broadcasted_iota(jnp.int32, (_B, _C, _C), 1)
    jj = jax.lax.broadcasted_iota(jnp.int32, (_B, _C, _C), 2)
    r = jnp.sum((gt | (eq & (jj < ii))).astype(jnp.int32), axis=2)

    match = (r[:, :, None] == perm_ref[...][0][None, None, :])
    ci = jax.lax.broadcasted_iota(jnp.int32, (_B, _C, _C), 1)
    sg = jnp.sum(jnp.where(match, ci, 0), axis=1)

    base = jax.lax.broadcasted_iota(jnp.int32, (_B, _B, _C), 0) * _C
    ind_ref[...] = base + sg[None, :, :]


def _gather_ring_body(sg_ref, x_ref, o_ref, xbuf, in_sem, out_sems):
    pltpu.make_async_copy(x_ref, xbuf, in_sem).start()

    def out_copy(k, slot):
        return pltpu.make_async_copy(
            xbuf.at[:, pl.ds(sg_ref[k], 1)],
            o_ref.at[:, pl.ds(k // _C, 1), pl.ds(jax.lax.rem(k, _C), 1)],
            out_sems.at[slot])

    pltpu.make_async_copy(x_ref, xbuf, in_sem).wait()

    def step(k, _):
        slot = jax.lax.rem(k, _N)

        @pl.when(k >= _N)
        def _ring():
            out_copy(k - _N, slot).wait()

        out_copy(k, slot).start()
        return 0

    jax.lax.fori_loop(0, _BC, step, 0, unroll=4)

    def drain(k, _):
        out_copy(k, jax.lax.rem(k, _N)).wait()
        return 0

    jax.lax.fori_loop(_BC - _N, _BC, drain, 0)


@jax.jit
def kernel(x, W1, b1, W2, b2):
    perm = _perm_const().astype(jnp.int32).reshape(1, _C)

    ind = pl.pallas_call(
        _index_body,
        out_shape=jax.ShapeDtypeStruct((_B, _B, _C), jnp.int32),
    )(x, W1, b1.reshape(1, _HID), W2, b2.reshape(1, _C), perm)

    sg_flat = ind[0].reshape(_BC)

    out = pl.pallas_call(
        _gather_ring_body,
        in_specs=[
            pl.BlockSpec(memory_space=pltpu.MemorySpace.SMEM),
            pl.BlockSpec(memory_space=pltpu.MemorySpace.HBM),
        ],
        out_specs=pl.BlockSpec(memory_space=pltpu.MemorySpace.HBM),
        out_shape=jax.ShapeDtypeStruct((_B, _B, _C, _H, _W), jnp.float32),
        scratch_shapes=[
            pltpu.VMEM((_B, _C, 1, _H, _W), jnp.float32),
            pltpu.SemaphoreType.DMA,
            pltpu.SemaphoreType.DMA((_N,)),
        ],
    )(sg_flat, x[:, :, None])

    return out


# speedup vs baseline: 1.6420x; 1.0200x over previous
import functools

import jax
import jax.numpy as jnp
from jax.experimental import pallas as pl
from jax.experimental.pallas import tpu as pltpu

_B, _C, _H, _W = 8, 96, 56, 56
_HW = _H * _W
_HID = _C // 16
_BC = _B * _C
_N = 8


def _perm_const():
    pkey = jax.random.key(42)
    return jax.random.permutation(jax.random.fold_in(pkey, 0), _C)


def _fused_body(x_ref, w1_ref, b1_ref, w2_ref, b2_ref, perm_ref, o_ref,
                xbuf, sg_vmem, sg_smem, in_sem, sg_sem, out_sems):
    pltpu.make_async_copy(x_ref, xbuf, in_sem).start()
    pltpu.make_async_copy(x_ref, xbuf, in_sem).wait()

    s = jnp.mean(xbuf[:, :, 0], axis=(2, 3))
    h = jnp.maximum(
        jax.lax.dot_general(s, w1_ref[...], (((1,), (1,)), ((), ())))
        + b1_ref[...], 0.0)
    lg = jax.lax.dot_general(h, w2_ref[...], (((1,), (1,)), ((), ())))
    sc = jax.nn.sigmoid(lg + b2_ref[...])

    gt = (sc[:, None, :] > sc[:, :, None])
    eq = (sc[:, None, :] == sc[:, :, None])
    ii = jax.lax.broadcasted_iota(jnp.int32, (_B, _C, _C), 1)
    jj = jax.lax.broadcasted_iota(jnp.int32, (_B, _C, _C), 2)
    r = jnp.sum((gt | (eq & (jj < ii))).astype(jnp.int32), axis=2)
    match = (r[:, :, None] == perm_ref[...][0][None, None, :])
    sg = jnp.sum(jnp.where(match, ii, 0), axis=1)

    sg_vmem[...] = sg
    pltpu.make_async_copy(sg_vmem, sg_smem, sg_sem).start()
    pltpu.make_async_copy(sg_vmem, sg_smem, sg_sem).wait()

    def out_copy(k, slot):
        b = k // _C
        j = jax.lax.rem(k, _C)
        return pltpu.make_async_copy(
            xbuf.at[:, pl.ds(sg_smem[b, j], 1)],
            o_ref.at[:, pl.ds(b, 1), pl.ds(j, 1)],
            out_sems.at[slot])

    def step(k, _):
        slot = jax.lax.rem(k, _N)

        @pl.when(k >= _N)
        def _ring():
            out_copy(k - _N, slot).wait()

        out_copy(k, slot).start()
        return 0

    jax.lax.fori_loop(0, _BC, step, 0, unroll=4)

    def drain(k, _):
        out_copy(k, jax.lax.rem(k, _N)).wait()
        return 0

    jax.lax.fori_loop(_BC - _N, _BC, drain, 0)


@jax.jit
def kernel(x, W1, b1, W2, b2):
    perm = _perm_const().astype(jnp.int32).reshape(1, _C)

    out = pl.pallas_call(
        _fused_body,
        in_specs=[
            pl.BlockSpec(memory_space=pltpu.MemorySpace.HBM),
            pl.BlockSpec(memory_space=pltpu.MemorySpace.VMEM),
            pl.BlockSpec(memory_space=pltpu.MemorySpace.VMEM),
            pl.BlockSpec(memory_space=pltpu.MemorySpace.VMEM),
            pl.BlockSpec(memory_space=pltpu.MemorySpace.VMEM),
            pl.BlockSpec(memory_space=pltpu.MemorySpace.VMEM),
        ],
        out_specs=pl.BlockSpec(memory_space=pltpu.MemorySpace.HBM),
        out_shape=jax.ShapeDtypeStruct((_B, _B, _C, _H, _W), jnp.float32),
        scratch_shapes=[
            pltpu.VMEM((_B, _C, 1, _H, _W), jnp.float32),
            pltpu.VMEM((_B, _C), jnp.int32),
            pltpu.SMEM((_B, _C), jnp.int32),
            pltpu.SemaphoreType.DMA,
            pltpu.SemaphoreType.DMA,
            pltpu.SemaphoreType.DMA((_N,)),
        ],
    )(x[:, :, None], W1, b1.reshape(1, _HID), W2, b2.reshape(1, _C), perm)

    return out


# speedup vs baseline: 1.7589x; 1.0712x over previous
import functools

import jax
import jax.numpy as jnp
from jax.experimental import pallas as pl
from jax.experimental.pallas import tpu as pltpu

_B, _C, _H, _W = 8, 96, 56, 56
_HW = _H * _W
_HID = _C // 16
_BC = _B * _C
_N = 16


def _perm_const():
    pkey = jax.random.key(42)
    return jax.random.permutation(jax.random.fold_in(pkey, 0), _C)


def _fused_body(x_ref, w1_ref, b1_ref, w2_ref, b2_ref, perm_ref, o_ref,
                xbuf, sg_vmem, sg_smem, in_sem, sg_sem, out_sems):
    pltpu.make_async_copy(x_ref, xbuf, in_sem).start()
    pltpu.make_async_copy(x_ref, xbuf, in_sem).wait()

    s = jnp.mean(xbuf[:, :, 0], axis=(2, 3))
    h = jnp.maximum(
        jax.lax.dot_general(s, w1_ref[...], (((1,), (1,)), ((), ())))
        + b1_ref[...], 0.0)
    lg = jax.lax.dot_general(h, w2_ref[...], (((1,), (1,)), ((), ())))
    sc = jax.nn.sigmoid(lg + b2_ref[...])

    gt = (sc[:, None, :] > sc[:, :, None])
    eq = (sc[:, None, :] == sc[:, :, None])
    ii = jax.lax.broadcasted_iota(jnp.int32, (_B, _C, _C), 1)
    jj = jax.lax.broadcasted_iota(jnp.int32, (_B, _C, _C), 2)
    r = jnp.sum((gt | (eq & (jj < ii))).astype(jnp.int32), axis=2)
    match = (r[:, :, None] == perm_ref[...][0][None, None, :])
    sg = jnp.sum(jnp.where(match, ii, 0), axis=1)

    sg_vmem[...] = sg
    pltpu.make_async_copy(sg_vmem, sg_smem, sg_sem).start()
    pltpu.make_async_copy(sg_vmem, sg_smem, sg_sem).wait()

    def out_copy(k, slot):
        b = k // _C
        j = jax.lax.rem(k, _C)
        return pltpu.make_async_copy(
            xbuf.at[:, pl.ds(sg_smem[b, j], 1)],
            o_ref.at[:, pl.ds(b, 1), pl.ds(j, 1)],
            out_sems.at[slot])

    def step(k, _):
        slot = jax.lax.rem(k, _N)

        @pl.when(k >= _N)
        def _ring():
            out_copy(k - _N, slot).wait()

        out_copy(k, slot).start()
        return 0

    jax.lax.fori_loop(0, _BC, step, 0, unroll=4)

    def drain(k, _):
        out_copy(k, jax.lax.rem(k, _N)).wait()
        return 0

    jax.lax.fori_loop(_BC - _N, _BC, drain, 0)


@jax.jit
def kernel(x, W1, b1, W2, b2):
    perm = _perm_const().astype(jnp.int32).reshape(1, _C)

    out = pl.pallas_call(
        _fused_body,
        in_specs=[
            pl.BlockSpec(memory_space=pltpu.MemorySpace.HBM),
            pl.BlockSpec(memory_space=pltpu.MemorySpace.VMEM),
            pl.BlockSpec(memory_space=pltpu.MemorySpace.VMEM),
            pl.BlockSpec(memory_space=pltpu.MemorySpace.VMEM),
            pl.BlockSpec(memory_space=pltpu.MemorySpace.VMEM),
            pl.BlockSpec(memory_space=pltpu.MemorySpace.VMEM),
        ],
        out_specs=pl.BlockSpec(memory_space=pltpu.MemorySpace.HBM),
        out_shape=jax.ShapeDtypeStruct((_B, _B, _C, _H, _W), jnp.float32),
        scratch_shapes=[
            pltpu.VMEM((_B, _C, 1, _H, _W), jnp.float32),
            pltpu.VMEM((_B, _C), jnp.int32),
            pltpu.SMEM((_B, _C), jnp.int32),
            pltpu.SemaphoreType.DMA,
            pltpu.SemaphoreType.DMA,
            pltpu.SemaphoreType.DMA((_N,)),
        ],
    )(x[:, :, None], W1, b1.reshape(1, _HID), W2, b2.reshape(1, _C), perm)

    return out


# speedup vs baseline: 1.7597x; 1.0005x over previous
import functools

import jax
import jax.numpy as jnp
from jax.experimental import pallas as pl
from jax.experimental.pallas import tpu as pltpu

_B, _C, _H, _W = 8, 96, 56, 56
_HW = _H * _W
_HID = _C // 16
_BC = _B * _C
_N = 32


def _perm_const():
    pkey = jax.random.key(42)
    return jax.random.permutation(jax.random.fold_in(pkey, 0), _C)


def _fused_body(x_ref, w1_ref, b1_ref, w2_ref, b2_ref, perm_ref, o_ref,
                xbuf, sg_vmem, sg_smem, in_sem, sg_sem, out_sems):
    pltpu.make_async_copy(x_ref, xbuf, in_sem).start()
    pltpu.make_async_copy(x_ref, xbuf, in_sem).wait()

    s = jnp.mean(xbuf[:, :, 0], axis=(2, 3))
    h = jnp.maximum(
        jax.lax.dot_general(s, w1_ref[...], (((1,), (1,)), ((), ())))
        + b1_ref[...], 0.0)
    lg = jax.lax.dot_general(h, w2_ref[...], (((1,), (1,)), ((), ())))
    sc = jax.nn.sigmoid(lg + b2_ref[...])

    gt = (sc[:, None, :] > sc[:, :, None])
    eq = (sc[:, None, :] == sc[:, :, None])
    ii = jax.lax.broadcasted_iota(jnp.int32, (_B, _C, _C), 1)
    jj = jax.lax.broadcasted_iota(jnp.int32, (_B, _C, _C), 2)
    r = jnp.sum((gt | (eq & (jj < ii))).astype(jnp.int32), axis=2)
    match = (r[:, :, None] == perm_ref[...][0][None, None, :])
    sg = jnp.sum(jnp.where(match, ii, 0), axis=1)

    sg_vmem[...] = sg
    pltpu.make_async_copy(sg_vmem, sg_smem, sg_sem).start()
    pltpu.make_async_copy(sg_vmem, sg_smem, sg_sem).wait()

    def out_copy(k, slot):
        b = k // _C
        j = jax.lax.rem(k, _C)
        return pltpu.make_async_copy(
            xbuf.at[:, pl.ds(sg_smem[b, j], 1)],
            o_ref.at[:, pl.ds(b, 1), pl.ds(j, 1)],
            out_sems.at[slot])

    def step(k, _):
        slot = jax.lax.rem(k, _N)

        @pl.when(k >= _N)
        def _ring():
            out_copy(k - _N, slot).wait()

        out_copy(k, slot).start()
        return 0

    jax.lax.fori_loop(0, _BC, step, 0, unroll=8)

    def drain(k, _):
        out_copy(k, jax.lax.rem(k, _N)).wait()
        return 0

    jax.lax.fori_loop(_BC - _N, _BC, drain, 0)


@jax.jit
def kernel(x, W1, b1, W2, b2):
    perm = _perm_const().astype(jnp.int32).reshape(1, _C)

    out = pl.pallas_call(
        _fused_body,
        in_specs=[
            pl.BlockSpec(memory_space=pltpu.MemorySpace.HBM),
            pl.BlockSpec(memory_space=pltpu.MemorySpace.VMEM),
            pl.BlockSpec(memory_space=pltpu.MemorySpace.VMEM),
            pl.BlockSpec(memory_space=pltpu.MemorySpace.VMEM),
            pl.BlockSpec(memory_space=pltpu.MemorySpace.VMEM),
            pl.BlockSpec(memory_space=pltpu.MemorySpace.VMEM),
        ],
        out_specs=pl.BlockSpec(memory_space=pltpu.MemorySpace.HBM),
        out_shape=jax.ShapeDtypeStruct((_B, _B, _C, _H, _W), jnp.float32),
        scratch_shapes=[
            pltpu.VMEM((_B, _C, 1, _H, _W), jnp.float32),
            pltpu.VMEM((_B, _C), jnp.int32),
            pltpu.SMEM((_B, _C), jnp.int32),
            pltpu.SemaphoreType.DMA,
            pltpu.SemaphoreType.DMA,
            pltpu.SemaphoreType.DMA((_N,)),
        ],
    )(x[:, :, None], W1, b1.reshape(1, _HID), W2, b2.reshape(1, _C), perm)

    return out
